# baseline (speedup 1.0000x reference)
import functools

import jax
import jax.numpy as jnp
from jax import lax
from jax.experimental import pallas as pl
from jax.experimental.pallas import tpu as pltpu
from jax.experimental.pallas import tpu_sc as plsc

_VOL = 128 * 128 * 128
_HALF = _VOL // 2
_NS = 16
_NC = 2
_PPT = _VOL // _NS
_CH = 2048
_NCHUNK = _PPT // _CH
_SLICE = _HALF // _NS
_ZB = 8192

_ROWS = 2048
_COLS = 1024
_BR = 256


def _sc_body(vals_hbm, w_hbm, idx_hbm, out_hbm,
             idx_v, w_v, v_v, lin_s, val_s, zbuf, spmem):
    c = lax.axis_index("c")
    s = lax.axis_index("s")
    iota = lax.iota(jnp.int32, 16)
    zf16 = jnp.zeros((16,), jnp.float32)
    is_w = lax.broadcast(c, (16,)) == 0
    base = pl.multiple_of(s * _SLICE, 8)

    def _zz(i, _):
        zbuf[pl.ds(i * 16, 16)] = zf16
        return 0

    for p in range(2):
        lax.fori_loop(jnp.int32(0), jnp.int32(_ZB // 16), _zz, 0)
        for j in range(_SLICE // _ZB):
            pltpu.sync_copy(zbuf, spmem.at[pl.ds(pl.multiple_of(base + j * _ZB, 8), _ZB)])
        plsc.subcore_barrier()

        def _chunk(it, _):
            p0 = s * _PPT + it * _CH
            pltpu.sync_copy(idx_hbm.at[pl.ds(pl.multiple_of(p0 * 6, 8), _CH * 6)], idx_v)
            pltpu.sync_copy(w_hbm.at[pl.ds(pl.multiple_of(p0, 8), _CH)], w_v)
            pltpu.sync_copy(vals_hbm.at[pl.ds(pl.multiple_of(p0 // 8, 8), _CH // 8)], v_v)

            def _vec(k, _):
                g = k * 16 + iota
                ix = plsc.load_gather(idx_v, [g * 6])
                iy = plsc.load_gather(idx_v, [g * 6 + 2])
                iz = plsc.load_gather(idx_v, [g * 6 + 4])
                lin = (ix * 128 + iy) * 128 + iz - jnp.int32(p * _HALF)
                w16 = w_v[pl.ds(k * 16, 16)]
                vg = plsc.load_gather(v_v, [lax.shift_right_logical(g, jnp.int32(3))])
                val = jnp.where(is_w, w16, w16 * vg)
                owned = (lin >= 0) & (lin < _HALF)
                kr = lax.shift_right_logical(k, jnp.int32(3))
                kc = (k & 7) * 16
                lin_s[kr, pl.ds(kc, 16)] = jnp.where(owned, lin, 0)
                val_s[kr, pl.ds(kc, 16)] = jnp.where(owned, val, 0.0)
                return 0
            lax.fori_loop(jnp.int32(0), jnp.int32(_CH // 16), _vec, 0)
            for j in range(_CH // 128):
                pltpu.sync_copy(val_s.at[jnp.int32(j)],
                                spmem.at[lin_s.at[jnp.int32(j)]], add=True)
            return 0
        lax.fori_loop(jnp.int32(0), jnp.int32(_NCHUNK), _chunk, 0)
        plsc.subcore_barrier()

        for j in range(_SLICE // _ZB):
            off = pl.multiple_of(base + j * _ZB, 8)
            hoff = pl.multiple_of(base + j * _ZB + p * _HALF, 8)
            pltpu.sync_copy(spmem.at[pl.ds(off, _ZB)], zbuf)
            pltpu.sync_copy(zbuf, out_hbm.at[c, pl.ds(hoff, _ZB)])


_sc_scatter = functools.partial(
    pl.kernel,
    out_type=[jax.ShapeDtypeStruct((_NC, _VOL), jnp.float32)],
    compiler_params=pltpu.CompilerParams(needs_layout_passes=False),
    mesh=plsc.VectorSubcoreMesh(
        core_axis_name="c", subcore_axis_name="s",
        num_cores=_NC, num_subcores=_NS),
    scratch_types=[
        pltpu.VMEM((_CH * 6,), jnp.int32),
        pltpu.VMEM((_CH,), jnp.float32),
        pltpu.VMEM((_CH // 8,), jnp.float32),
        pltpu.VMEM((_CH // 128, 128), jnp.int32),
        pltpu.VMEM((_CH // 128, 128), jnp.float32),
        pltpu.VMEM((_ZB,), jnp.float32),
        pltpu.MemorySpace.VMEM_SHARED((_HALF,), jnp.float32),
    ],
)(_sc_body)


def _combine_body(wc_ref, vc_ref, wold_ref, vold_ref, outv_ref, outw_ref):
    wc = wc_ref[0]
    vc = vc_ref[0]
    w_old = wold_ref[...]
    v_old = vold_ref[...]
    touched = wc > 0.0
    denom = w_old + wc
    safe = jnp.where(touched, denom, 1.0)
    new_v = (w_old * v_old + vc) / safe
    outv_ref[...] = jnp.where(touched, new_v, v_old)
    outw_ref[...] = jnp.where(touched, denom, w_old)


def _combine(caches, values_volume, weights_volume):
    shp = values_volume.shape
    c3 = caches.reshape(_NC, _ROWS, _COLS)
    args = [c3, c3] + [
        a.reshape(_ROWS, _COLS)
        for a in (weights_volume, values_volume)
    ]
    spec = pl.BlockSpec((_BR, _COLS), lambda i, j: (i, j))
    wspec = pl.BlockSpec((1, _BR, _COLS), lambda i, j: (i * 0, i, j))
    vspec = pl.BlockSpec((1, _BR, _COLS), lambda i, j: (i * 0 + 1, i, j))
    out_v, out_w = pl.pallas_call(
        _combine_body,
        grid=(_ROWS // _BR, 1),
        in_specs=[wspec, vspec, spec, spec],
        out_specs=[spec, spec],
        out_shape=[
            jax.ShapeDtypeStruct((_ROWS, _COLS), jnp.float32),
            jax.ShapeDtypeStruct((_ROWS, _COLS), jnp.float32),
        ],
    )(*args)
    return out_v.reshape(shp), out_w.reshape(shp)


def kernel(values, indices, weights, values_volume, weights_volume):
    n8 = weights.size
    idx6 = lax.bitcast_convert_type(
        indices.reshape(n8, 3), jnp.int32).reshape(n8 * 6)
    vals = values.reshape(values.size)
    w = weights.reshape(n8)
    caches, = _sc_scatter(vals, w, idx6)
    return _combine(caches, values_volume, weights_volume)


# speedup vs baseline: 1.5299x; 1.5299x over previous
import functools

import jax
import jax.numpy as jnp
from jax import lax
from jax.experimental import pallas as pl
from jax.experimental.pallas import tpu as pltpu
from jax.experimental.pallas import tpu_sc as plsc

_VOL = 128 * 128 * 128
_HALF = _VOL // 2
_NS = 16
_NC = 2
_PPT = _VOL // _NS
_CH = 2048
_NCHUNK = _PPT // _CH
_SLICE = _HALF // _NS
_ZB = 8192

_ROWS = 2048
_COLS = 1024
_BR = 256


def _sc_body(vals_hbm, w_hbm, idx_hbm, out_hbm,
             idx_v, w_v, v_v, lin_s, val_s, zbuf, spmem, sem):
    c = lax.axis_index("c")
    s = lax.axis_index("s")
    iota = lax.iota(jnp.int32, 16)
    zf16 = jnp.zeros((16,), jnp.float32)
    is_w = lax.broadcast(c, (16,)) == 0
    base = pl.multiple_of(s * _SLICE, 8)

    def _zz(i, _):
        zbuf[pl.ds(i * 16, 16)] = zf16
        return 0

    for p in range(2):
        lax.fori_loop(jnp.int32(0), jnp.int32(_ZB // 16), _zz, 0)
        for j in range(_SLICE // _ZB):
            pltpu.sync_copy(zbuf, spmem.at[pl.ds(pl.multiple_of(base + j * _ZB, 8), _ZB)])
        plsc.subcore_barrier()

        def _chunk(it, _):
            p0 = s * _PPT + it * _CH
            pltpu.sync_copy(idx_hbm.at[pl.ds(pl.multiple_of(p0 * 3, 8), _CH * 3)], idx_v)
            pltpu.sync_copy(w_hbm.at[pl.ds(pl.multiple_of(p0, 8), _CH)], w_v)
            pltpu.sync_copy(vals_hbm.at[pl.ds(pl.multiple_of(p0 // 8, 8), _CH // 8)], v_v)

            def _vec(k, _):
                g = k * 16 + iota
                ix = plsc.load_gather(idx_v, [g * 3])
                iy = plsc.load_gather(idx_v, [g * 3 + 1])
                iz = plsc.load_gather(idx_v, [g * 3 + 2])
                lin = (ix * 128 + iy) * 128 + iz - jnp.int32(p * _HALF)
                w16 = w_v[pl.ds(k * 16, 16)]
                vg = plsc.load_gather(v_v, [lax.shift_right_logical(g, jnp.int32(3))])
                val = jnp.where(is_w, w16, w16 * vg)
                owned = (lin >= 0) & (lin < _HALF)
                kr = lax.shift_right_logical(k, jnp.int32(3))
                kc = (k & 7) * 16
                lin_s[kr, pl.ds(kc, 16)] = jnp.where(owned, lin, 0)
                val_s[kr, pl.ds(kc, 16)] = jnp.where(owned, val, 0.0)
                return 0
            lax.fori_loop(jnp.int32(0), jnp.int32(_CH // 16), _vec, 0)
            descs = [
                pltpu.async_copy(val_s.at[jnp.int32(j)],
                                 spmem.at[lin_s.at[jnp.int32(j)]], sem,
                                 add=True)
                for j in range(_CH // 128)
            ]
            for d in descs:
                d.wait()
            return 0
        lax.fori_loop(jnp.int32(0), jnp.int32(_NCHUNK), _chunk, 0)
        plsc.subcore_barrier()

        for j in range(_SLICE // _ZB):
            off = pl.multiple_of(base + j * _ZB, 8)
            hoff = pl.multiple_of(base + j * _ZB + p * _HALF, 8)
            pltpu.sync_copy(spmem.at[pl.ds(off, _ZB)], zbuf)
            pltpu.sync_copy(zbuf, out_hbm.at[c, pl.ds(hoff, _ZB)])


_sc_scatter = functools.partial(
    pl.kernel,
    out_type=[jax.ShapeDtypeStruct((_NC, _VOL), jnp.float32)],
    compiler_params=pltpu.CompilerParams(needs_layout_passes=False),
    mesh=plsc.VectorSubcoreMesh(
        core_axis_name="c", subcore_axis_name="s",
        num_cores=_NC, num_subcores=_NS),
    scratch_types=[
        pltpu.VMEM((_CH * 3,), jnp.int32),
        pltpu.VMEM((_CH,), jnp.float32),
        pltpu.VMEM((_CH // 8,), jnp.float32),
        pltpu.VMEM((_CH // 128, 128), jnp.int32),
        pltpu.VMEM((_CH // 128, 128), jnp.float32),
        pltpu.VMEM((_ZB,), jnp.float32),
        pltpu.MemorySpace.VMEM_SHARED((_HALF,), jnp.float32),
        pltpu.SemaphoreType.DMA,
    ],
)(_sc_body)


def _combine_body(wc_ref, vc_ref, wold_ref, vold_ref, outv_ref, outw_ref):
    wc = wc_ref[0]
    vc = vc_ref[0]
    w_old = wold_ref[...]
    v_old = vold_ref[...]
    touched = wc > 0.0
    denom = w_old + wc
    safe = jnp.where(touched, denom, 1.0)
    new_v = (w_old * v_old + vc) / safe
    outv_ref[...] = jnp.where(touched, new_v, v_old)
    outw_ref[...] = jnp.where(touched, denom, w_old)


def _combine(caches, values_volume, weights_volume):
    shp = values_volume.shape
    c3 = caches.reshape(_NC, _ROWS, _COLS)
    args = [c3, c3] + [
        a.reshape(_ROWS, _COLS)
        for a in (weights_volume, values_volume)
    ]
    spec = pl.BlockSpec((_BR, _COLS), lambda i, j: (i, j))
    wspec = pl.BlockSpec((1, _BR, _COLS), lambda i, j: (i * 0, i, j))
    vspec = pl.BlockSpec((1, _BR, _COLS), lambda i, j: (i * 0 + 1, i, j))
    out_v, out_w = pl.pallas_call(
        _combine_body,
        grid=(_ROWS // _BR, 1),
        in_specs=[wspec, vspec, spec, spec],
        out_specs=[spec, spec],
        out_shape=[
            jax.ShapeDtypeStruct((_ROWS, _COLS), jnp.float32),
            jax.ShapeDtypeStruct((_ROWS, _COLS), jnp.float32),
        ],
    )(*args)
    return out_v.reshape(shp), out_w.reshape(shp)


def kernel(values, indices, weights, values_volume, weights_volume):
    n8 = weights.size
    idx3 = indices.astype(jnp.int32).reshape(n8 * 3)
    vals = values.reshape(values.size)
    w = weights.reshape(n8)
    caches, = _sc_scatter(vals, w, idx3)
    return _combine(caches, values_volume, weights_volume)


# speedup vs baseline: 1.9166x; 1.2527x over previous
import functools

import jax
import jax.numpy as jnp
from jax import lax
from jax.experimental import pallas as pl
from jax.experimental.pallas import tpu as pltpu
from jax.experimental.pallas import tpu_sc as plsc

_VOL = 128 * 128 * 128
_HALF = _VOL // 2
_NS = 16
_NC = 2
_PPT = _VOL // _NS
_CH = 2048
_NCHUNK = _PPT // _CH
_SLICE = _HALF // _NS
_ZB = 8192

_ROWS = 2048
_COLS = 1024
_BR = 256


def _sc_body(vals_hbm, w_hbm, idx_hbm, out_hbm,
             idx_v, w_v, v_v, lin_s, val_s, zbuf, spmem, sem):
    c = lax.axis_index("c")
    s = lax.axis_index("s")
    iota = lax.iota(jnp.int32, 16)
    zf16 = jnp.zeros((16,), jnp.float32)
    is_w = lax.broadcast(c, (16,)) == 0
    base = pl.multiple_of(s * _SLICE, 8)

    def _zz(i, _):
        zbuf[pl.ds(i * 16, 16)] = zf16
        return 0

    for p in range(2):
        lax.fori_loop(jnp.int32(0), jnp.int32(_ZB // 16), _zz, 0)
        for j in range(_SLICE // _ZB):
            pltpu.sync_copy(zbuf, spmem.at[pl.ds(pl.multiple_of(base + j * _ZB, 8), _ZB)])
        plsc.subcore_barrier()

        def _chunk(it, _):
            p0 = s * _PPT + it * _CH
            pltpu.sync_copy(idx_hbm.at[pl.ds(pl.multiple_of(p0 * 3, 8), _CH * 3)], idx_v)
            pltpu.sync_copy(w_hbm.at[pl.ds(pl.multiple_of(p0, 8), _CH)], w_v)
            pltpu.sync_copy(vals_hbm.at[pl.ds(pl.multiple_of(p0 // 8, 8), _CH // 8)], v_v)

            def _vec(k, _):
                g = k * 16 + iota
                ix = plsc.load_gather(idx_v, [g * 3])
                iy = plsc.load_gather(idx_v, [g * 3 + 1])
                iz = plsc.load_gather(idx_v, [g * 3 + 2])
                lin = (ix * 128 + iy) * 128 + iz - jnp.int32(p * _HALF)
                w16 = w_v[pl.ds(k * 16, 16)]
                vg = plsc.load_gather(v_v, [lax.shift_right_logical(g, jnp.int32(3))])
                val = jnp.where(is_w, w16, w16 * vg)
                owned = (lin >= 0) & (lin < _HALF)
                kr = lax.shift_right_logical(k, jnp.int32(3))
                kc = (k & 7) * 16
                lin_s[kr, pl.ds(kc, 16)] = jnp.where(owned, lin, g)
                val_s[kr, pl.ds(kc, 16)] = jnp.where(owned, val, 0.0)
                return 0
            lax.fori_loop(jnp.int32(0), jnp.int32(_CH // 16), _vec, 0)
            descs = [
                pltpu.async_copy(val_s.at[jnp.int32(j)],
                                 spmem.at[lin_s.at[jnp.int32(j)]], sem,
                                 add=True)
                for j in range(_CH // 128)
            ]
            for d in descs:
                d.wait()
            return 0
        lax.fori_loop(jnp.int32(0), jnp.int32(_NCHUNK), _chunk, 0)
        plsc.subcore_barrier()

        for j in range(_SLICE // _ZB):
            off = pl.multiple_of(base + j * _ZB, 8)
            hoff = pl.multiple_of(base + j * _ZB + p * _HALF, 8)
            pltpu.sync_copy(spmem.at[pl.ds(off, _ZB)], zbuf)
            pltpu.sync_copy(zbuf, out_hbm.at[c, pl.ds(hoff, _ZB)])


_sc_scatter = functools.partial(
    pl.kernel,
    out_type=[jax.ShapeDtypeStruct((_NC, _VOL), jnp.float32)],
    compiler_params=pltpu.CompilerParams(needs_layout_passes=False),
    mesh=plsc.VectorSubcoreMesh(
        core_axis_name="c", subcore_axis_name="s",
        num_cores=_NC, num_subcores=_NS),
    scratch_types=[
        pltpu.VMEM((_CH * 3,), jnp.int32),
        pltpu.VMEM((_CH,), jnp.float32),
        pltpu.VMEM((_CH // 8,), jnp.float32),
        pltpu.VMEM((_CH // 128, 128), jnp.int32),
        pltpu.VMEM((_CH // 128, 128), jnp.float32),
        pltpu.VMEM((_ZB,), jnp.float32),
        pltpu.MemorySpace.VMEM_SHARED((_HALF,), jnp.float32),
        pltpu.SemaphoreType.DMA,
    ],
)(_sc_body)


def _combine_body(wc_ref, vc_ref, wold_ref, vold_ref, outv_ref, outw_ref):
    wc = wc_ref[0]
    vc = vc_ref[0]
    w_old = wold_ref[...]
    v_old = vold_ref[...]
    touched = wc > 0.0
    denom = w_old + wc
    safe = jnp.where(touched, denom, 1.0)
    new_v = (w_old * v_old + vc) / safe
    outv_ref[...] = jnp.where(touched, new_v, v_old)
    outw_ref[...] = jnp.where(touched, denom, w_old)


def _combine(caches, values_volume, weights_volume):
    shp = values_volume.shape
    c3 = caches.reshape(_NC, _ROWS, _COLS)
    args = [c3, c3] + [
        a.reshape(_ROWS, _COLS)
        for a in (weights_volume, values_volume)
    ]
    spec = pl.BlockSpec((_BR, _COLS), lambda i, j: (i, j))
    wspec = pl.BlockSpec((1, _BR, _COLS), lambda i, j: (i * 0, i, j))
    vspec = pl.BlockSpec((1, _BR, _COLS), lambda i, j: (i * 0 + 1, i, j))
    out_v, out_w = pl.pallas_call(
        _combine_body,
        grid=(_ROWS // _BR, 1),
        in_specs=[wspec, vspec, spec, spec],
        out_specs=[spec, spec],
        out_shape=[
            jax.ShapeDtypeStruct((_ROWS, _COLS), jnp.float32),
            jax.ShapeDtypeStruct((_ROWS, _COLS), jnp.float32),
        ],
    )(*args)
    return out_v.reshape(shp), out_w.reshape(shp)


def kernel(values, indices, weights, values_volume, weights_volume):
    n8 = weights.size
    idx3 = indices.astype(jnp.int32).reshape(n8 * 3)
    vals = values.reshape(values.size)
    w = weights.reshape(n8)
    caches, = _sc_scatter(vals, w, idx3)
    return _combine(caches, values_volume, weights_volume)


# speedup vs baseline: 1.9186x; 1.0010x over previous
import functools

import jax
import jax.numpy as jnp
from jax import lax
from jax.experimental import pallas as pl
from jax.experimental.pallas import tpu as pltpu
from jax.experimental.pallas import tpu_sc as plsc

_VOL = 128 * 128 * 128
_HALF = _VOL // 2
_NS = 16
_NC = 2
_PPT = _VOL // _NS
_CH = 2048
_NCHUNK = _PPT // _CH
_SLICE = _HALF // _NS
_ZB = 8192

_ROWS = 2048
_COLS = 1024
_BR = 256


def _sc_body(vals_hbm, w_hbm, idx_hbm, out_hbm,
             idx_v, w_v, v_v, lin_s, val_s, zbuf, spmem, sem):
    c = lax.axis_index("c")
    s = lax.axis_index("s")
    iota = lax.iota(jnp.int32, 16)
    zf16 = jnp.zeros((16,), jnp.float32)
    is_w = lax.broadcast(c, (16,)) == 0
    base = pl.multiple_of(s * _SLICE, 8)

    def _zz(i, _):
        zbuf[pl.ds(i * 16, 16)] = zf16
        return 0

    for p in range(2):
        lax.fori_loop(jnp.int32(0), jnp.int32(_ZB // 16), _zz, 0)
        for j in range(_SLICE // _ZB):
            pltpu.sync_copy(zbuf, spmem.at[pl.ds(pl.multiple_of(base + j * _ZB, 8), _ZB)])
        plsc.subcore_barrier()

        def _chunk(it, _):
            p0 = s * _PPT + it * _CH
            pltpu.sync_copy(idx_hbm.at[pl.ds(pl.multiple_of(p0 * 3, 8), _CH * 3)], idx_v)
            pltpu.sync_copy(w_hbm.at[pl.ds(pl.multiple_of(p0, 8), _CH)], w_v)
            pltpu.sync_copy(vals_hbm.at[pl.ds(pl.multiple_of(p0 // 8, 8), _CH // 8)], v_v)

            def _vec(k, _):
                g = k * 16 + iota
                ix = plsc.load_gather(idx_v, [g * 3])
                iy = plsc.load_gather(idx_v, [g * 3 + 1])
                iz = plsc.load_gather(idx_v, [g * 3 + 2])
                lin = (ix * 128 + iy) * 128 + iz - jnp.int32(p * _HALF)
                w16 = w_v[pl.ds(k * 16, 16)]
                vg = plsc.load_gather(v_v, [lax.shift_right_logical(g, jnp.int32(3))])
                val = jnp.where(is_w, w16, w16 * vg)
                owned = (lin >= 0) & (lin < _HALF)
                kr = lax.shift_right_logical(k, jnp.int32(3))
                kc = (k & 7) * 16
                lin_s[kr, pl.ds(kc, 16)] = lin & jnp.int32(_HALF - 1)
                val_s[kr, pl.ds(kc, 16)] = jnp.where(owned, val, 0.0)
                return 0
            lax.fori_loop(jnp.int32(0), jnp.int32(_CH // 16), _vec, 0)
            descs = [
                pltpu.async_copy(val_s.at[jnp.int32(j)],
                                 spmem.at[lin_s.at[jnp.int32(j)]], sem,
                                 add=True)
                for j in range(_CH // 128)
            ]
            for d in descs:
                d.wait()
            return 0
        lax.fori_loop(jnp.int32(0), jnp.int32(_NCHUNK), _chunk, 0)
        plsc.subcore_barrier()

        for j in range(_SLICE // _ZB):
            off = pl.multiple_of(base + j * _ZB, 8)
            hoff = pl.multiple_of(base + j * _ZB + p * _HALF, 8)
            pltpu.sync_copy(spmem.at[pl.ds(off, _ZB)], zbuf)
            pltpu.sync_copy(zbuf, out_hbm.at[c, pl.ds(hoff, _ZB)])


_sc_scatter = functools.partial(
    pl.kernel,
    out_type=[jax.ShapeDtypeStruct((_NC, _VOL), jnp.float32)],
    compiler_params=pltpu.CompilerParams(needs_layout_passes=False),
    mesh=plsc.VectorSubcoreMesh(
        core_axis_name="c", subcore_axis_name="s",
        num_cores=_NC, num_subcores=_NS),
    scratch_types=[
        pltpu.VMEM((_CH * 3,), jnp.int32),
        pltpu.VMEM((_CH,), jnp.float32),
        pltpu.VMEM((_CH // 8,), jnp.float32),
        pltpu.VMEM((_CH // 128, 128), jnp.int32),
        pltpu.VMEM((_CH // 128, 128), jnp.float32),
        pltpu.VMEM((_ZB,), jnp.float32),
        pltpu.MemorySpace.VMEM_SHARED((_HALF,), jnp.float32),
        pltpu.SemaphoreType.DMA,
    ],
)(_sc_body)


def _combine_body(wc_ref, vc_ref, wold_ref, vold_ref, outv_ref, outw_ref):
    wc = wc_ref[0]
    vc = vc_ref[0]
    w_old = wold_ref[...]
    v_old = vold_ref[...]
    touched = wc > 0.0
    denom = w_old + wc
    safe = jnp.where(touched, denom, 1.0)
    new_v = (w_old * v_old + vc) / safe
    outv_ref[...] = jnp.where(touched, new_v, v_old)
    outw_ref[...] = jnp.where(touched, denom, w_old)


def _combine(caches, values_volume, weights_volume):
    shp = values_volume.shape
    c3 = caches.reshape(_NC, _ROWS, _COLS)
    args = [c3, c3] + [
        a.reshape(_ROWS, _COLS)
        for a in (weights_volume, values_volume)
    ]
    spec = pl.BlockSpec((_BR, _COLS), lambda i, j: (i, j))
    wspec = pl.BlockSpec((1, _BR, _COLS), lambda i, j: (i * 0, i, j))
    vspec = pl.BlockSpec((1, _BR, _COLS), lambda i, j: (i * 0 + 1, i, j))
    out_v, out_w = pl.pallas_call(
        _combine_body,
        grid=(_ROWS // _BR, 1),
        in_specs=[wspec, vspec, spec, spec],
        out_specs=[spec, spec],
        out_shape=[
            jax.ShapeDtypeStruct((_ROWS, _COLS), jnp.float32),
            jax.ShapeDtypeStruct((_ROWS, _COLS), jnp.float32),
        ],
    )(*args)
    return out_v.reshape(shp), out_w.reshape(shp)


def kernel(values, indices, weights, values_volume, weights_volume):
    n8 = weights.size
    idx3 = indices.astype(jnp.int32).reshape(n8 * 3)
    vals = values.reshape(values.size)
    w = weights.reshape(n8)
    caches, = _sc_scatter(vals, w, idx3)
    return _combine(caches, values_volume, weights_volume)
